# baseline (speedup 1.0000x reference)
import functools

import jax
import jax.numpy as jnp
from jax import lax
from jax.experimental import pallas as pl
from jax.experimental.pallas import tpu as pltpu
from jax.experimental.pallas import tpu_sc as plsc

NUM_CORES = 2
NUM_SUBCORES = 16
NW = NUM_CORES * NUM_SUBCORES
L = 16

CHUNK = 128
EPS = 1e-5


def _rsqrt(v):
    i = lax.bitcast_convert_type(v, jnp.int32)
    y = lax.bitcast_convert_type(jnp.int32(0x5F3759DF) - (i >> 1), jnp.float32)
    for _ in range(3):
        y = y * (1.5 - 0.5 * v * y * y)
    return y


def _make_transpose_kernel(v, d, v_main, blk):
    n_blocks = v_main // blk
    mesh = plsc.VectorSubcoreMesh(
        core_axis_name="c", subcore_axis_name="s",
        num_cores=NUM_CORES, num_subcores=NUM_SUBCORES)

    @functools.partial(
        pl.kernel,
        out_type=jax.ShapeDtypeStruct((v * d,), jnp.float32),
        mesh=mesh,
        scratch_types=[
            pltpu.VMEM((d, blk), jnp.float32),
            pltpu.VMEM((d, blk), jnp.float32),
            pltpu.VMEM((blk * d,), jnp.float32),
            pltpu.VMEM((blk * d,), jnp.float32),
            pltpu.VMEM(((v - v_main) * d,), jnp.float32),
            pltpu.SemaphoreType.DMA,
            pltpu.SemaphoreType.DMA,
        ],
        compiler_params=pltpu.CompilerParams(
            needs_layout_passes=False, use_tc_tiling_on_sc=True),
    )
    def tr_kernel(tbl_t_hbm, tail_hbm, out_hbm, inb0, inb1, outb0, outb1,
                  tailb, s0, s1):
        wid = lax.axis_index("s") * NUM_CORES + lax.axis_index("c")
        inbs = (inb0, inb1)
        outbs = (outb0, outb1)
        gsems = (s0, s1)
        iota_d = lax.iota(jnp.int32, L) * d

        @pl.when(wid == 0)
        def _():
            pltpu.sync_copy(tail_hbm, tailb)
            pltpu.sync_copy(tailb, out_hbm.at[pl.ds(v_main * d, (v - v_main) * d)])

        def stage(i, s):
            pltpu.async_copy(
                tbl_t_hbm.at[:, pl.ds((wid + i * NW) * blk, blk)],
                inbs[s], gsems[s])

        my_blocks = n_blocks // NW + jnp.where(wid < n_blocks % NW, 1, 0)
        nb_max = n_blocks // NW + (1 if n_blocks % NW else 0)

        @pl.when(my_blocks > 0)
        def _():
            stage(0, 0)

        def do_block(i, s):
            @pl.when(i < my_blocks)
            def _():
                @pl.when(i + 1 < my_blocks)
                def _():
                    stage(i + 1, 1 - s)
                pltpu.make_async_copy(
                    tbl_t_hbm.at[:, pl.ds((wid + i * NW) * blk, blk)],
                    inbs[s], gsems[s]).wait()

                def col_body(c, _):
                    base = iota_d + c
                    for g in range(blk // L):
                        vals = inbs[s][c, pl.ds(g * L, L)]
                        plsc.store_scatter(
                            outbs[s], [base + g * (L * d)], vals)
                    return 0

                lax.fori_loop(0, d, col_body, 0)
                pltpu.sync_copy(
                    outbs[s],
                    out_hbm.at[pl.ds((wid + i * NW) * blk * d, blk * d)])

        def outer(it, _):
            do_block(it * 2, 0)
            do_block(it * 2 + 1, 1)
            return 0

        lax.fori_loop(0, (nb_max + 1) // 2, outer, 0)

    return tr_kernel


def _make_kernel(n_rows, d, t_len, n_batch):
    rows_pw = n_rows // NW
    n_chunks = rows_pw // CHUNK
    nv = d // L

    mesh = plsc.VectorSubcoreMesh(
        core_axis_name="c", subcore_axis_name="s",
        num_cores=NUM_CORES, num_subcores=NUM_SUBCORES)

    @functools.partial(
        pl.kernel,
        out_type=jax.ShapeDtypeStruct((n_rows, d), jnp.float32),
        mesh=mesh,
        scratch_types=[
            pltpu.VMEM((2, CHUNK), jnp.int32),
            pltpu.VMEM((2, CHUNK), jnp.int32),
            pltpu.VMEM((2, CHUNK, 64), jnp.float32),
            pltpu.VMEM((2, CHUNK, 64), jnp.float32),
            pltpu.VMEM((t_len, d), jnp.float32),
            pltpu.VMEM((d,), jnp.float32),
            pltpu.VMEM((d,), jnp.float32),
            pltpu.SemaphoreType.DMA,
            pltpu.SemaphoreType.DMA,
            pltpu.SemaphoreType.DMA,
            pltpu.SemaphoreType.DMA,
        ],
        compiler_params=pltpu.CompilerParams(
            needs_layout_passes=False, use_tc_tiling_on_sc=False),
    )
    def emb_kernel(ids_hbm, tok_hbm, pos_hbm, gamma_hbm, beta_hbm, out_hbm,
                   idx_v, didx_v, gbuf, obuf, pos_v, g_v, b_v,
                   gsem0, gsem1, osem0, osem1):
        wid = lax.axis_index("s") * NUM_CORES + lax.axis_index("c")
        w_base = wid * rows_pw
        gsems = (gsem0, gsem1)
        osems = (osem0, osem1)
        iota200 = lax.iota(jnp.int32, L) * t_len

        pltpu.sync_copy(pos_hbm.at[pl.ds(0, t_len)], pos_v)
        pltpu.sync_copy(gamma_hbm, g_v)
        pltpu.sync_copy(beta_hbm, b_v)
        gs = [g_v[pl.ds(i * L, L)] for i in range(nv)]
        bs = [b_v[pl.ds(i * L, L)] for i in range(nv)]

        def stage_gather(c, s):
            pltpu.sync_copy(ids_hbm.at[pl.ds(w_base + c * CHUNK, CHUNK)],
                            idx_v.at[s])
            pltpu.async_copy(tok_hbm.at[idx_v.at[s]], gbuf.at[s], gsems[s])

        stage_gather(0, 0)
        stage_gather(1, 1)

        def chunk_body(c, s):
            f0 = w_base + c * CHUNK
            t = f0 >> 10
            b0 = f0 & (n_batch - 1)
            prow = [pos_v[t, pl.ds(i * L, L)] for i in range(nv)]

            pltpu.make_async_copy(
                tok_hbm.at[idx_v.at[s]], gbuf.at[s], gsems[s]).wait()

            @pl.when(c >= 2)
            def _():
                pltpu.make_async_copy(
                    obuf.at[s], out_hbm.at[didx_v.at[s]], osems[s]).wait()

            d0 = b0 * t_len + t
            for j in range(CHUNK // L):
                didx_v[s, pl.ds(j * L, L)] = iota200 + (d0 + j * L * t_len)

            def row_body(r, _):
                xs = []
                for i in range(nv):
                    xs.append(gbuf[s, r, pl.ds(i * L, L)] + prow[i])
                sv = xs[0]
                qv = xs[0] * xs[0]
                for x in xs[1:]:
                    sv = sv + x
                    qv = qv + x * x
                mean = jnp.sum(sv) * (1.0 / d)
                var = jnp.sum(qv) * (1.0 / d) - mean * mean
                rstd = _rsqrt(var + EPS)
                for i in range(nv):
                    obuf[s, r, pl.ds(i * L, L)] = \
                        (xs[i] - mean) * rstd * gs[i] + bs[i]
                return 0

            lax.fori_loop(0, CHUNK, row_body, 0)

            pltpu.async_copy(obuf.at[s], out_hbm.at[didx_v.at[s]], osems[s])

            @pl.when(c + 2 < n_chunks)
            def _():
                stage_gather(c + 2, s)
            return s

        def outer(it, _):
            chunk_body(it * 2, 0)
            chunk_body(it * 2 + 1, 1)
            return 0

        lax.fori_loop(0, n_chunks // 2, outer, 0)

        pltpu.make_async_copy(
            obuf.at[0], out_hbm.at[didx_v.at[0]], osems[0]).wait()
        pltpu.make_async_copy(
            obuf.at[1], out_hbm.at[didx_v.at[1]], osems[1]).wait()

    return emb_kernel


@jax.jit
def kernel(ids, token_table, pos_table, gamma, beta):
    b, t = ids.shape
    v, d = token_table.shape
    blk = 256
    v_main = (v // blk) * blk
    tr = _make_transpose_kernel(v, d, v_main, blk)
    tail = token_table[v_main:, :].reshape(-1)
    tok_lin = tr(token_table.T, tail).reshape(v, d)
    ids2 = ids.T.reshape(-1).astype(jnp.int32)
    emb = _make_kernel(b * t, d, t, b)
    out_flat = emb(ids2, tok_lin, pos_table, gamma, beta)
    return out_flat.reshape(b, t, d)


# speedup vs baseline: 1.6462x; 1.6462x over previous
import functools

import jax
import jax.numpy as jnp
from jax import lax
from jax.experimental import pallas as pl
from jax.experimental.pallas import tpu as pltpu
from jax.experimental.pallas import tpu_sc as plsc

NUM_CORES = 2
NUM_SUBCORES = 16
NW = NUM_CORES * NUM_SUBCORES
L = 16

CHUNK = 128
EPS = 1e-5


def _rsqrt(v):
    i = lax.bitcast_convert_type(v, jnp.int32)
    y = lax.bitcast_convert_type(jnp.int32(0x5F3759DF) - (i >> 1), jnp.float32)
    for _ in range(3):
        y = y * (1.5 - 0.5 * v * y * y)
    return y


def _make_transpose_kernel(v, d, v_main, blk):
    n_blocks = v_main // blk
    mesh = plsc.VectorSubcoreMesh(
        core_axis_name="c", subcore_axis_name="s",
        num_cores=NUM_CORES, num_subcores=NUM_SUBCORES)

    @functools.partial(
        pl.kernel,
        out_type=jax.ShapeDtypeStruct((v * d,), jnp.float32),
        mesh=mesh,
        scratch_types=[
            pltpu.VMEM((d * blk,), jnp.float32),
            pltpu.VMEM((d * blk,), jnp.float32),
            pltpu.VMEM((blk * d,), jnp.float32),
            pltpu.VMEM((blk * d,), jnp.float32),
            pltpu.VMEM(((v - v_main) * d,), jnp.float32),
            pltpu.SemaphoreType.DMA,
            pltpu.SemaphoreType.DMA,
        ],
        compiler_params=pltpu.CompilerParams(
            needs_layout_passes=False, use_tc_tiling_on_sc=True),
    )
    def tr_kernel(tbl_t_hbm, tail_hbm, out_hbm, inb0, inb1, outb0, outb1,
                  tailb, s0, s1):
        wid = lax.axis_index("s") * NUM_CORES + lax.axis_index("c")
        inbs = (inb0, inb1)
        outbs = (outb0, outb1)
        gsems = (s0, s1)
        iota = lax.iota(jnp.int32, L)

        @pl.when(wid == 0)
        def _():
            pltpu.sync_copy(tail_hbm, tailb)
            pltpu.sync_copy(tailb, out_hbm.at[pl.ds(v_main * d, (v - v_main) * d)])

        def stage(i, s):
            for c in range(d):
                pltpu.async_copy(
                    tbl_t_hbm.at[c, pl.ds((wid + i * NW) * blk, blk)],
                    inbs[s].at[pl.ds(c * blk, blk)], gsems[s])

        def wait_stage(i, s):
            for c in range(d):
                pltpu.make_async_copy(
                    tbl_t_hbm.at[c, pl.ds((wid + i * NW) * blk, blk)],
                    inbs[s].at[pl.ds(c * blk, blk)], gsems[s]).wait()

        my_blocks = n_blocks // NW + jnp.where(wid < n_blocks % NW, 1, 0)
        nb_max = n_blocks // NW + (1 if n_blocks % NW else 0)

        @pl.when(my_blocks > 0)
        def _():
            stage(0, 0)

        n_tiles = (d // L) * (blk // L)

        def do_block(i, s):
            @pl.when(i < my_blocks)
            def _():
                @pl.when(i + 1 < my_blocks)
                def _():
                    stage(i + 1, 1 - s)
                wait_stage(i, s)

                for j in range(L):
                    rot = (iota + j) & (L - 1)
                    pc = rot * blk + iota
                    po = iota * d + rot

                    def tile_body(t, _, pc=pc, po=po):
                        cb = t >> 4
                        g = t & 15
                        idx_in = pc + (cb * (L * blk) + g * L)
                        idx_out = po + (g * (L * d) + cb * L)
                        vals = plsc.load_gather(inbs[s], [idx_in])
                        plsc.store_scatter(outbs[s], [idx_out], vals)
                        return 0

                    lax.fori_loop(0, n_tiles, tile_body, 0)

                pltpu.sync_copy(
                    outbs[s],
                    out_hbm.at[pl.ds((wid + i * NW) * blk * d, blk * d)])

        def outer(it, _):
            do_block(it * 2, 0)
            do_block(it * 2 + 1, 1)
            return 0

        lax.fori_loop(0, (nb_max + 1) // 2, outer, 0)

    return tr_kernel


def _make_kernel(n_rows, d, t_len, n_batch):
    rows_pw = n_rows // NW
    n_chunks = rows_pw // CHUNK
    nv = d // L

    mesh = plsc.VectorSubcoreMesh(
        core_axis_name="c", subcore_axis_name="s",
        num_cores=NUM_CORES, num_subcores=NUM_SUBCORES)

    @functools.partial(
        pl.kernel,
        out_type=jax.ShapeDtypeStruct((n_rows, d), jnp.float32),
        mesh=mesh,
        scratch_types=[
            pltpu.VMEM((2, CHUNK), jnp.int32),
            pltpu.VMEM((2, CHUNK), jnp.int32),
            pltpu.VMEM((2, CHUNK, 64), jnp.float32),
            pltpu.VMEM((2, CHUNK, 64), jnp.float32),
            pltpu.VMEM((t_len, d), jnp.float32),
            pltpu.VMEM((d,), jnp.float32),
            pltpu.VMEM((d,), jnp.float32),
            pltpu.SemaphoreType.DMA,
            pltpu.SemaphoreType.DMA,
            pltpu.SemaphoreType.DMA,
            pltpu.SemaphoreType.DMA,
        ],
        compiler_params=pltpu.CompilerParams(
            needs_layout_passes=False, use_tc_tiling_on_sc=False),
    )
    def emb_kernel(ids_hbm, tok_hbm, pos_hbm, gamma_hbm, beta_hbm, out_hbm,
                   idx_v, didx_v, gbuf, obuf, pos_v, g_v, b_v,
                   gsem0, gsem1, osem0, osem1):
        wid = lax.axis_index("s") * NUM_CORES + lax.axis_index("c")
        w_base = wid * rows_pw
        gsems = (gsem0, gsem1)
        osems = (osem0, osem1)
        iota200 = lax.iota(jnp.int32, L) * t_len

        pltpu.sync_copy(pos_hbm.at[pl.ds(0, t_len)], pos_v)
        pltpu.sync_copy(gamma_hbm, g_v)
        pltpu.sync_copy(beta_hbm, b_v)
        gs = [g_v[pl.ds(i * L, L)] for i in range(nv)]
        bs = [b_v[pl.ds(i * L, L)] for i in range(nv)]

        def stage_gather(c, s):
            pltpu.sync_copy(ids_hbm.at[pl.ds(w_base + c * CHUNK, CHUNK)],
                            idx_v.at[s])
            pltpu.async_copy(tok_hbm.at[idx_v.at[s]], gbuf.at[s], gsems[s])

        stage_gather(0, 0)
        stage_gather(1, 1)

        def chunk_body(c, s):
            f0 = w_base + c * CHUNK
            t = f0 >> 10
            b0 = f0 & (n_batch - 1)
            prow = [pos_v[t, pl.ds(i * L, L)] for i in range(nv)]

            pltpu.make_async_copy(
                tok_hbm.at[idx_v.at[s]], gbuf.at[s], gsems[s]).wait()

            @pl.when(c >= 2)
            def _():
                pltpu.make_async_copy(
                    obuf.at[s], out_hbm.at[didx_v.at[s]], osems[s]).wait()

            d0 = b0 * t_len + t
            for j in range(CHUNK // L):
                didx_v[s, pl.ds(j * L, L)] = iota200 + (d0 + j * L * t_len)

            def row_body(r, _):
                xs = []
                for i in range(nv):
                    xs.append(gbuf[s, r, pl.ds(i * L, L)] + prow[i])
                sv = xs[0]
                qv = xs[0] * xs[0]
                for x in xs[1:]:
                    sv = sv + x
                    qv = qv + x * x
                mean = jnp.sum(sv) * (1.0 / d)
                var = jnp.sum(qv) * (1.0 / d) - mean * mean
                rstd = _rsqrt(var + EPS)
                for i in range(nv):
                    obuf[s, r, pl.ds(i * L, L)] = \
                        (xs[i] - mean) * rstd * gs[i] + bs[i]
                return 0

            lax.fori_loop(0, CHUNK, row_body, 0)

            pltpu.async_copy(obuf.at[s], out_hbm.at[didx_v.at[s]], osems[s])

            @pl.when(c + 2 < n_chunks)
            def _():
                stage_gather(c + 2, s)
            return s

        def outer(it, _):
            chunk_body(it * 2, 0)
            chunk_body(it * 2 + 1, 1)
            return 0

        lax.fori_loop(0, n_chunks // 2, outer, 0)

        pltpu.make_async_copy(
            obuf.at[0], out_hbm.at[didx_v.at[0]], osems[0]).wait()
        pltpu.make_async_copy(
            obuf.at[1], out_hbm.at[didx_v.at[1]], osems[1]).wait()

    return emb_kernel


@jax.jit
def kernel(ids, token_table, pos_table, gamma, beta):
    b, t = ids.shape
    v, d = token_table.shape
    blk = 256
    v_main = (v // blk) * blk
    tr = _make_transpose_kernel(v, d, v_main, blk)
    tail = token_table[v_main:, :].reshape(-1)
    tok_lin = tr(token_table.T, tail).reshape(v, d)
    ids2 = ids.T.reshape(-1).astype(jnp.int32)
    emb = _make_kernel(b * t, d, t, b)
    out_flat = emb(ids2, tok_lin, pos_table, gamma, beta)
    return out_flat.reshape(b, t, d)


# speedup vs baseline: 2.0039x; 1.2173x over previous
import functools

import jax
import jax.numpy as jnp
from jax import lax
from jax.experimental import layout as jax_layout
from jax.experimental import pallas as pl
from jax.experimental.pallas import tpu as pltpu
from jax.experimental.pallas import tpu_sc as plsc

NUM_CORES = 2
NUM_SUBCORES = 16
NW = NUM_CORES * NUM_SUBCORES
L = 16

CHUNK = 128
EPS = 1e-5


def _rsqrt(v):
    i = lax.bitcast_convert_type(v, jnp.int32)
    y = lax.bitcast_convert_type(jnp.int32(0x5F3759DF) - (i >> 1), jnp.float32)
    for _ in range(3):
        y = y * (1.5 - 0.5 * v * y * y)
    return y


def _make_kernel(n_rows, d, t_len, n_batch):
    rows_pw = n_rows // NW
    n_chunks = rows_pw // CHUNK
    nv = d // L

    mesh = plsc.VectorSubcoreMesh(
        core_axis_name="c", subcore_axis_name="s",
        num_cores=NUM_CORES, num_subcores=NUM_SUBCORES)

    @functools.partial(
        pl.kernel,
        out_type=jax.ShapeDtypeStruct((n_rows, d), jnp.float32),
        mesh=mesh,
        scratch_types=[
            pltpu.VMEM((2, CHUNK), jnp.int32),
            pltpu.VMEM((2, CHUNK), jnp.int32),
            pltpu.VMEM((2, CHUNK, 64), jnp.float32),
            pltpu.VMEM((2, CHUNK, 64), jnp.float32),
            pltpu.VMEM((t_len, d), jnp.float32),
            pltpu.VMEM((d,), jnp.float32),
            pltpu.VMEM((d,), jnp.float32),
            pltpu.SemaphoreType.DMA,
            pltpu.SemaphoreType.DMA,
            pltpu.SemaphoreType.DMA,
            pltpu.SemaphoreType.DMA,
        ],
        compiler_params=pltpu.CompilerParams(
            needs_layout_passes=False, use_tc_tiling_on_sc=False),
    )
    def emb_kernel(ids_hbm, tok_hbm, pos_hbm, gamma_hbm, beta_hbm, out_hbm,
                   idx_v, didx_v, gbuf, obuf, pos_v, g_v, b_v,
                   gsem0, gsem1, osem0, osem1):
        wid = lax.axis_index("s") * NUM_CORES + lax.axis_index("c")
        w_base = wid * rows_pw
        gsems = (gsem0, gsem1)
        osems = (osem0, osem1)
        iota200 = lax.iota(jnp.int32, L) * t_len

        pltpu.sync_copy(pos_hbm.at[pl.ds(0, t_len)], pos_v)
        pltpu.sync_copy(gamma_hbm, g_v)
        pltpu.sync_copy(beta_hbm, b_v)
        gs = [g_v[pl.ds(i * L, L)] for i in range(nv)]
        bs = [b_v[pl.ds(i * L, L)] for i in range(nv)]

        def stage_gather(c, s):
            pltpu.sync_copy(ids_hbm.at[pl.ds(w_base + c * CHUNK, CHUNK)],
                            idx_v.at[s])
            pltpu.async_copy(tok_hbm.at[idx_v.at[s]], gbuf.at[s], gsems[s])

        stage_gather(0, 0)
        stage_gather(1, 1)

        def chunk_body(c, s):
            f0 = w_base + c * CHUNK
            t = f0 >> 10
            b0 = f0 & (n_batch - 1)
            prow = [pos_v[t, pl.ds(i * L, L)] for i in range(nv)]

            pltpu.make_async_copy(
                tok_hbm.at[idx_v.at[s]], gbuf.at[s], gsems[s]).wait()

            @pl.when(c >= 2)
            def _():
                pltpu.make_async_copy(
                    obuf.at[s], out_hbm.at[didx_v.at[s]], osems[s]).wait()

            d0 = b0 * t_len + t
            for j in range(CHUNK // L):
                didx_v[s, pl.ds(j * L, L)] = iota200 + (d0 + j * L * t_len)

            def row_body(r, _):
                xs = []
                for i in range(nv):
                    xs.append(gbuf[s, r, pl.ds(i * L, L)] + prow[i])
                sv = xs[0]
                qv = xs[0] * xs[0]
                for x in xs[1:]:
                    sv = sv + x
                    qv = qv + x * x
                mean = jnp.sum(sv) * (1.0 / d)
                var = jnp.sum(qv) * (1.0 / d) - mean * mean
                rstd = _rsqrt(var + EPS)
                for i in range(nv):
                    obuf[s, r, pl.ds(i * L, L)] = \
                        (xs[i] - mean) * rstd * gs[i] + bs[i]
                return 0

            lax.fori_loop(0, CHUNK, row_body, 0)

            pltpu.async_copy(obuf.at[s], out_hbm.at[didx_v.at[s]], osems[s])

            @pl.when(c + 2 < n_chunks)
            def _():
                stage_gather(c + 2, s)
            return s

        def outer(it, _):
            chunk_body(it * 2, 0)
            chunk_body(it * 2 + 1, 1)
            return 0

        lax.fori_loop(0, n_chunks // 2, outer, 0)

        pltpu.make_async_copy(
            obuf.at[0], out_hbm.at[didx_v.at[0]], osems[0]).wait()
        pltpu.make_async_copy(
            obuf.at[1], out_hbm.at[didx_v.at[1]], osems[1]).wait()

    return emb_kernel


def _kernel_impl(ids, token_table, pos_table, gamma, beta):
    b, t = ids.shape
    v, d = token_table.shape
    tok2 = jnp.pad(token_table, ((0, 0), (0, 64))).reshape(2 * v, d)
    ids2 = (ids.T.reshape(-1) * 2).astype(jnp.int32)
    emb = _make_kernel(b * t, d, t, b)
    out_flat = emb(ids2, tok2, pos_table, gamma, beta)
    return out_flat.reshape(b, t, d)


_jitted = None


def kernel(ids, token_table, pos_table, gamma, beta):
    global _jitted
    if _jitted is None:
        fmt = jax_layout.Format(
            jax_layout.Layout(major_to_minor=(0, 1, 2), tiling=((8, 128),)),
            jax.sharding.SingleDeviceSharding(jax.devices()[0]))
        _jitted = jax.jit(_kernel_impl, out_shardings=fmt)
    return _jitted(ids, token_table, pos_table, gamma, beta)
